# baseline (speedup 1.0000x reference)
import functools

import jax
import jax.numpy as jnp
from jax import lax
from jax.experimental import pallas as pl
from jax.experimental.pallas import tpu as pltpu
from jax.experimental.pallas import tpu_sc as plsc

_NE = 1024
_D = 64
_B = 32
_L = 576
_N = _B * _L
_CC = 0.1

_NW = 32
_BPW = _N // _NW
_CHUNKS = 6
_CG = _BPW // _CHUNKS


def _vq_body(x_ref, e_ref, e2_ref, jcol_ref, idx_ref, loss_ref,
             perp_ref, cnt_ref, kl_ref):
    b = pl.program_id(0)

    @pl.when(b == 0)
    def _init():
        cnt_ref[...] = jnp.zeros_like(cnt_ref)
        kl_ref[...] = jnp.zeros_like(kl_ref)

    x = x_ref[0]
    e = e_ref[...]
    e2 = e2_ref[...]
    x2 = jnp.sum(x * x, axis=0)
    scores_m2 = jax.lax.dot_general(
        e * -2.0, x, dimension_numbers=(((1,), (0,)), ((), ())),
        preferred_element_type=jnp.float32)
    dist = (x2[None, :] + e2) + scores_m2
    jcol = jcol_ref[...]
    m = jnp.min(dist, axis=0)
    idx_f = jnp.min(jnp.where(dist == m[None, :], jcol, float(_NE)),
                    axis=0)
    idx = idx_f.astype(jnp.int32)

    onehot = (jcol == idx_f[None, :]).astype(jnp.float32)
    q = jax.lax.dot_general(
        e, onehot, dimension_numbers=(((0,), (0,)), ((), ())),
        preferred_element_type=jnp.float32)

    idx_ref[0, 0] = idx
    cnt_ref[...] += jnp.sum(onehot, axis=1, keepdims=True)

    sm_x = jax.nn.softmax(x, axis=0)
    sm_q = jax.nn.softmax(q, axis=0)
    kl_ref[...] += jnp.sum(sm_x * (jnp.log(sm_x) - sm_q)).reshape(1, 1)

    @pl.when(b == _B - 1)
    def _fin():
        loss_ref[...] = _CC * kl_ref[...] / _B
        p = cnt_ref[...] / _N
        perp_ref[...] = jnp.exp(-jnp.sum(p * jnp.log(p + 1e-10))).reshape(1, 1)


def _vq_call(inputs, embedding_weight, e2, jcol):
    return pl.pallas_call(
        _vq_body,
        grid=(_B,),
        in_specs=[
            pl.BlockSpec((1, _D, _L), lambda b: (b, 0, 0)),
            pl.BlockSpec((_NE, _D), lambda b: (0, 0)),
            pl.BlockSpec((_NE, 1), lambda b: (0, 0)),
            pl.BlockSpec((_NE, 1), lambda b: (0, 0)),
        ],
        out_specs=[
            pl.BlockSpec((1, 1, _L), lambda b: (b, 0, 0)),
            pl.BlockSpec((1, 1), lambda b: (0, 0)),
            pl.BlockSpec((1, 1), lambda b: (0, 0)),
        ],
        out_shape=[
            jax.ShapeDtypeStruct((_B, 1, _L), jnp.int32),
            jax.ShapeDtypeStruct((1, 1), jnp.float32),
            jax.ShapeDtypeStruct((1, 1), jnp.float32),
        ],
        scratch_shapes=[
            pltpu.VMEM((_NE, 1), jnp.float32),
            pltpu.VMEM((1, 1), jnp.float32),
        ],
    )(inputs, embedding_weight, e2, jcol)


@functools.partial(
    pl.kernel,
    out_type=jax.ShapeDtypeStruct((_N, _D), jnp.float32),
    mesh=plsc.VectorSubcoreMesh(core_axis_name="c", subcore_axis_name="s"),
    compiler_params=pltpu.CompilerParams(use_tc_tiling_on_sc=False),
    scratch_types=[
        pltpu.VMEM((_BPW,), jnp.int32),
        pltpu.VMEM((_BPW, _D), jnp.float32),
        pltpu.SemaphoreType.DMA,
    ],
)
def _sc_gather_call(table_hbm, idx_hbm, out_hbm, idx_v, rows_v, sem):
    wid = lax.axis_index("s") * 2 + lax.axis_index("c")
    pltpu.sync_copy(idx_hbm.at[pl.ds(wid * _BPW, _BPW)], idx_v)
    copies = [
        pltpu.async_copy(table_hbm.at[idx_v.at[pl.ds(k * _CG, _CG)]],
                         rows_v.at[pl.ds(k * _CG, _CG)], sem)
        for k in range(_CHUNKS)
    ]
    for cp in copies:
        cp.wait()
    pltpu.sync_copy(rows_v, out_hbm.at[pl.ds(wid * _BPW, _BPW)])


def kernel(inputs, embedding_weight):
    e2 = jnp.sum(embedding_weight ** 2, axis=1)[:, None]
    jcol = jnp.arange(_NE, dtype=jnp.float32)[:, None]
    idx, loss, perp = _vq_call(inputs, embedding_weight, e2, jcol)
    rows = _sc_gather_call(embedding_weight, idx.reshape(_N))
    out = jnp.transpose(rows.reshape(_B, _L, _D), (0, 2, 1))
    return (out, loss[0, 0], perp[0, 0], embedding_weight,
            idx.reshape(_N, 1))


# speedup vs baseline: 1.9663x; 1.9663x over previous
import jax
import jax.numpy as jnp
from jax.experimental import pallas as pl
from jax.experimental.pallas import tpu as pltpu

_NE = 1024
_D = 64
_B = 32
_L = 576
_N = _B * _L
_CC = 0.1
_BB = 4
_G = _B // _BB


def _vq_body(x_ref, e_ref, e2_ref, jcol_ref, out_ref, idx_ref, loss_ref,
             perp_ref, cnt_ref, kl_ref):
    g = pl.program_id(0)

    @pl.when(g == 0)
    def _init():
        cnt_ref[...] = jnp.zeros_like(cnt_ref)
        kl_ref[...] = jnp.zeros_like(kl_ref)

    e = e_ref[...]
    e2 = e2_ref[...]
    jcol = jcol_ref[...]
    em2 = e * -2.0
    for s in range(_BB):
        x = x_ref[s]
        x2 = jnp.sum(x * x, axis=0)
        scores_m2 = jax.lax.dot_general(
            em2, x, dimension_numbers=(((1,), (0,)), ((), ())),
            preferred_element_type=jnp.float32)
        dist = (x2[None, :] + e2) + scores_m2
        m = jnp.min(dist, axis=0)
        idx_f = jnp.min(jnp.where(dist == m[None, :], jcol, float(_NE)),
                        axis=0)
        idx = idx_f.astype(jnp.int32)

        onehot = (jcol == idx_f[None, :]).astype(jnp.float32)
        q = jax.lax.dot_general(
            e, onehot, dimension_numbers=(((0,), (0,)), ((), ())),
            preferred_element_type=jnp.float32)

        out_ref[s] = x + (q - x)
        idx_ref[s, 0] = idx
        cnt_ref[...] += jnp.sum(onehot, axis=1, keepdims=True)

        sm_x = jax.nn.softmax(x, axis=0)
        sm_q = jax.nn.softmax(q, axis=0)
        kl_ref[...] += jnp.sum(sm_x * (jnp.log(sm_x) - sm_q)).reshape(1, 1)

    @pl.when(g == _G - 1)
    def _fin():
        loss_ref[...] = _CC * kl_ref[...] / _B
        p = cnt_ref[...] / _N
        perp_ref[...] = jnp.exp(-jnp.sum(p * jnp.log(p + 1e-10))).reshape(1, 1)


def _vq_call(inputs, embedding_weight, e2, jcol, interpret=False):
    return pl.pallas_call(
        _vq_body,
        grid=(_G,),
        in_specs=[
            pl.BlockSpec((_BB, _D, _L), lambda g: (g, 0, 0)),
            pl.BlockSpec((_NE, _D), lambda g: (0, 0)),
            pl.BlockSpec((_NE, 1), lambda g: (0, 0)),
            pl.BlockSpec((_NE, 1), lambda g: (0, 0)),
        ],
        out_specs=[
            pl.BlockSpec((_BB, _D, _L), lambda g: (g, 0, 0)),
            pl.BlockSpec((_BB, 1, _L), lambda g: (g, 0, 0)),
            pl.BlockSpec((1, 1), lambda g: (0, 0)),
            pl.BlockSpec((1, 1), lambda g: (0, 0)),
        ],
        out_shape=[
            jax.ShapeDtypeStruct((_B, _D, _L), jnp.float32),
            jax.ShapeDtypeStruct((_B, 1, _L), jnp.int32),
            jax.ShapeDtypeStruct((1, 1), jnp.float32),
            jax.ShapeDtypeStruct((1, 1), jnp.float32),
        ],
        scratch_shapes=[
            pltpu.VMEM((_NE, 1), jnp.float32),
            pltpu.VMEM((1, 1), jnp.float32),
        ],
        interpret=interpret,
    )(inputs, embedding_weight, e2, jcol)


def kernel(inputs, embedding_weight):
    e2 = jnp.sum(embedding_weight ** 2, axis=1)[:, None]
    jcol = jnp.arange(_NE, dtype=jnp.float32)[:, None]
    out, idx, loss, perp = _vq_call(inputs, embedding_weight, e2, jcol)
    return (out, loss[0, 0], perp[0, 0], embedding_weight,
            idx.reshape(_N, 1))


# speedup vs baseline: 1.9690x; 1.0014x over previous
import jax
import jax.numpy as jnp
from jax.experimental import pallas as pl
from jax.experimental.pallas import tpu as pltpu

_NE = 1024
_D = 64
_B = 32
_L = 576
_N = _B * _L
_CC = 0.1
_BB = 8
_G = _B // _BB


def _vq_body(x_ref, e_ref, e2_ref, jcol_ref, out_ref, idx_ref, loss_ref,
             perp_ref, cnt_ref, kl_ref):
    g = pl.program_id(0)

    @pl.when(g == 0)
    def _init():
        cnt_ref[...] = jnp.zeros_like(cnt_ref)
        kl_ref[...] = jnp.zeros_like(kl_ref)

    e = e_ref[...]
    e2 = e2_ref[...]
    jcol = jcol_ref[...]
    em2 = e * -2.0
    for s in range(_BB):
        x = x_ref[s]
        x2 = jnp.sum(x * x, axis=0)
        scores_m2 = jax.lax.dot_general(
            em2, x, dimension_numbers=(((1,), (0,)), ((), ())),
            preferred_element_type=jnp.float32)
        dist = (x2[None, :] + e2) + scores_m2
        m = jnp.min(dist, axis=0)
        idx_f = jnp.min(jnp.where(dist == m[None, :], jcol, float(_NE)),
                        axis=0)
        idx = idx_f.astype(jnp.int32)

        onehot = (jcol == idx_f[None, :]).astype(jnp.float32)
        q = jax.lax.dot_general(
            e, onehot, dimension_numbers=(((0,), (0,)), ((), ())),
            preferred_element_type=jnp.float32)

        out_ref[s] = x + (q - x)
        idx_ref[s, 0] = idx
        cnt_ref[...] += jnp.sum(onehot, axis=1, keepdims=True)

        sm_x = jax.nn.softmax(x, axis=0)
        sm_q = jax.nn.softmax(q, axis=0)
        kl_ref[...] += jnp.sum(sm_x * (jnp.log(sm_x) - sm_q)).reshape(1, 1)

    @pl.when(g == _G - 1)
    def _fin():
        loss_ref[...] = _CC * kl_ref[...] / _B
        p = cnt_ref[...] / _N
        perp_ref[...] = jnp.exp(-jnp.sum(p * jnp.log(p + 1e-10))).reshape(1, 1)


def _vq_call(inputs, embedding_weight, e2, jcol, interpret=False):
    return pl.pallas_call(
        _vq_body,
        grid=(_G,),
        in_specs=[
            pl.BlockSpec((_BB, _D, _L), lambda g: (g, 0, 0)),
            pl.BlockSpec((_NE, _D), lambda g: (0, 0)),
            pl.BlockSpec((_NE, 1), lambda g: (0, 0)),
            pl.BlockSpec((_NE, 1), lambda g: (0, 0)),
        ],
        out_specs=[
            pl.BlockSpec((_BB, _D, _L), lambda g: (g, 0, 0)),
            pl.BlockSpec((_BB, 1, _L), lambda g: (g, 0, 0)),
            pl.BlockSpec((1, 1), lambda g: (0, 0)),
            pl.BlockSpec((1, 1), lambda g: (0, 0)),
        ],
        out_shape=[
            jax.ShapeDtypeStruct((_B, _D, _L), jnp.float32),
            jax.ShapeDtypeStruct((_B, 1, _L), jnp.int32),
            jax.ShapeDtypeStruct((1, 1), jnp.float32),
            jax.ShapeDtypeStruct((1, 1), jnp.float32),
        ],
        scratch_shapes=[
            pltpu.VMEM((_NE, 1), jnp.float32),
            pltpu.VMEM((1, 1), jnp.float32),
        ],
        interpret=interpret,
    )(inputs, embedding_weight, e2, jcol)


def kernel(inputs, embedding_weight):
    e2 = jnp.sum(embedding_weight ** 2, axis=1)[:, None]
    jcol = jnp.arange(_NE, dtype=jnp.float32)[:, None]
    out, idx, loss, perp = _vq_call(inputs, embedding_weight, e2, jcol)
    return (out, loss[0, 0], perp[0, 0], embedding_weight,
            idx.reshape(_N, 1))
